# baseline (speedup 1.0000x reference)
import functools
import jax
import jax.numpy as jnp
from jax import lax
from jax.experimental import pallas as pl
from jax.experimental.pallas import tpu as pltpu
from jax.experimental.pallas import tpu_sc as plsc

BATCH = 16384
EMB = 32
NUM_CORES = 2
NUM_SUBCORES = 16
NUM_WORKERS = NUM_CORES * NUM_SUBCORES
BPW = BATCH // NUM_WORKERS
CHUNK = 128
NCHUNK = BPW // CHUNK


def _mf_body(u_hbm, v_hbm, ue_hbm, ie_hbm, out_hbm,
             uidx, vidx, urows, vrows, outb, sem):
    wid = lax.axis_index("s") * NUM_CORES + lax.axis_index("c")
    base = wid * BPW

    pltpu.sync_copy(u_hbm.at[wid], uidx)
    pltpu.sync_copy(v_hbm.at[wid], vidx)

    copies = []
    for g in range(NCHUNK):
        copies.append(pltpu.async_copy(
            ue_hbm.at[uidx.at[g]], urows.at[pl.ds(g * CHUNK, CHUNK)], sem))
        copies.append(pltpu.async_copy(
            ie_hbm.at[vidx.at[g]], vrows.at[pl.ds(g * CHUNK, CHUNK)], sem))
    for cp in copies:
        cp.wait()

    riota = lax.iota(jnp.int32, 16)

    def body(i, carry):
        rows16 = i * 16 + riota
        acc = jnp.zeros((16,), jnp.float32)
        for c in range(EMB):
            cvec = jnp.full((16,), c, jnp.int32)
            uc = plsc.load_gather(urows, [rows16, cvec])
            vc = plsc.load_gather(vrows, [rows16, cvec])
            acc = acc + uc * vc
        outb[pl.ds(pl.multiple_of(i * 16, 16), 16)] = acc
        return carry

    lax.fori_loop(0, BPW // 16, body, 0)

    pltpu.sync_copy(outb, out_hbm.at[pl.ds(base, BPW)])


_mf_kernel = functools.partial(
    pl.kernel,
    mesh=plsc.VectorSubcoreMesh(core_axis_name="c", subcore_axis_name="s"),
    out_type=jax.ShapeDtypeStruct((BATCH,), jnp.float32),
    scratch_types=[
        pltpu.VMEM((NCHUNK, CHUNK), jnp.int32),
        pltpu.VMEM((NCHUNK, CHUNK), jnp.int32),
        pltpu.VMEM((BPW, EMB), jnp.float32),
        pltpu.VMEM((BPW, EMB), jnp.float32),
        pltpu.VMEM((BPW,), jnp.float32),
        pltpu.SemaphoreType.DMA,
    ],
    compiler_params=pltpu.CompilerParams(
        needs_layout_passes=False, use_tc_tiling_on_sc=False),
)(_mf_body)


@jax.jit
def kernel(u, v, user_emb, item_emb):
    u3 = u.astype(jnp.int32).reshape(NUM_WORKERS, NCHUNK, CHUNK)
    v3 = v.astype(jnp.int32).reshape(NUM_WORKERS, NCHUNK, CHUNK)
    return _mf_kernel(u3, v3, user_emb, item_emb)


# speedup vs baseline: 1.4813x; 1.4813x over previous
import functools
import jax
import jax.numpy as jnp
from jax import lax
from jax.experimental import pallas as pl
from jax.experimental.pallas import tpu as pltpu
from jax.experimental.pallas import tpu_sc as plsc

BATCH = 16384
EMB = 32
NUM_CORES = 2
NUM_SUBCORES = 16
NUM_WORKERS = NUM_CORES * NUM_SUBCORES
BPW = BATCH // NUM_WORKERS
CH = 256
NCH = BPW // CH


def _mf_body(u_hbm, v_hbm, ue_hbm, ie_hbm, out_hbm,
             uidx, vidx, urows, vrows, outb, sem):
    wid = lax.axis_index("s") * NUM_CORES + lax.axis_index("c")
    base = wid * BPW

    pltpu.sync_copy(u_hbm.at[wid], uidx)
    pltpu.sync_copy(v_hbm.at[wid], vidx)

    riota = lax.iota(jnp.int32, 16)

    def chunk(h, carry):
        def fetch(i, c2):
            off = h * CH + i * 16
            uvec = uidx[pl.ds(pl.multiple_of(off, 16), 16)]
            vvec = vidx[pl.ds(pl.multiple_of(off, 16), 16)]
            for j in range(16):
                r = i * 16 + j
                pltpu.async_copy(ue_hbm.at[pl.ds(uvec[j], 1)],
                                 urows.at[pl.ds(r, 1)], sem)
                pltpu.async_copy(ie_hbm.at[pl.ds(vvec[j], 1)],
                                 vrows.at[pl.ds(r, 1)], sem)
            return c2

        lax.fori_loop(0, CH // 16, fetch, 0)

        pltpu.make_async_copy(ue_hbm.at[pl.ds(0, CH)], urows, sem).wait()
        pltpu.make_async_copy(ie_hbm.at[pl.ds(0, CH)], vrows, sem).wait()

        def body(i, c2):
            rows16 = i * 16 + riota
            acc = jnp.zeros((16,), jnp.float32)
            for c in range(EMB):
                cvec = jnp.full((16,), c, jnp.int32)
                uc = plsc.load_gather(urows, [rows16, cvec])
                vc = plsc.load_gather(vrows, [rows16, cvec])
                acc = acc + uc * vc
            outb[pl.ds(pl.multiple_of(h * CH + i * 16, 16), 16)] = acc
            return c2

        lax.fori_loop(0, CH // 16, body, 0)
        return carry

    lax.fori_loop(0, NCH, chunk, 0)

    pltpu.sync_copy(outb, out_hbm.at[pl.ds(base, BPW)])


_mf_kernel = functools.partial(
    pl.kernel,
    mesh=plsc.VectorSubcoreMesh(core_axis_name="c", subcore_axis_name="s"),
    out_type=jax.ShapeDtypeStruct((BATCH,), jnp.float32),
    scratch_types=[
        pltpu.VMEM((BPW,), jnp.int32),
        pltpu.VMEM((BPW,), jnp.int32),
        pltpu.VMEM((CH, EMB), jnp.float32),
        pltpu.VMEM((CH, EMB), jnp.float32),
        pltpu.VMEM((BPW,), jnp.float32),
        pltpu.SemaphoreType.DMA,
    ],
    compiler_params=pltpu.CompilerParams(needs_layout_passes=False),
)(_mf_body)


@jax.jit
def kernel(u, v, user_emb, item_emb):
    u2 = u.astype(jnp.int32).reshape(NUM_WORKERS, BPW)
    v2 = v.astype(jnp.int32).reshape(NUM_WORKERS, BPW)
    return _mf_kernel(u2, v2, user_emb, item_emb)
